# baseline (speedup 1.0000x reference)
import functools

import jax
import jax.numpy as jnp
from jax.experimental import pallas as pl


TILE_I = 512


def _split_bf16(x):
    hi = x.astype(jnp.bfloat16)
    lo = (x - hi.astype(jnp.float32)).astype(jnp.bfloat16)
    return hi, lo


def _chamfer_body(lhs_ref, rhs_ref, minp_ref, mint_ref):
    it = pl.program_id(1)
    g = jnp.dot(lhs_ref[0], rhs_ref[0],
                preferred_element_type=jnp.float32)
    minp_ref[0, 0, :] = jnp.min(g, axis=1)
    colmin = jnp.min(g, axis=0, keepdims=True)

    @pl.when(it == 0)
    def _init():
        mint_ref[0] = colmin

    @pl.when(it != 0)
    def _acc():
        mint_ref[0] = jnp.minimum(mint_ref[0], colmin)


@functools.partial(jax.jit, static_argnames=("interpret",))
def kernel(pred, target, interpret=False):
    B, N, _ = pred.shape
    M = target.shape[1]
    f32 = jnp.float32

    ph, plo = _split_bf16(pred)
    s = -2.0 * target
    sh, slo = _split_bf16(s)
    pn = jnp.sum(pred.astype(f32) ** 2, axis=2, keepdims=True)
    pnh, pnl = _split_bf16(pn)
    tn = jnp.sum(target.astype(f32) ** 2, axis=2, keepdims=True)
    tnh, tnl = _split_bf16(tn)
    one = jnp.ones((B, N, 1), jnp.bfloat16)
    lhs = jnp.concatenate(
        [ph, ph, plo, plo, pnh, pnl, one, one], axis=2
    ).astype(jnp.float32)
    onet = jnp.ones((B, M, 1), jnp.bfloat16)
    rhs_rows = jnp.concatenate(
        [sh, slo, sh, slo, onet, onet, tnh, tnl], axis=2)
    rhs = jnp.swapaxes(rhs_rows, 1, 2).astype(jnp.float32)

    grid = (B, N // TILE_I)
    minp, mint = pl.pallas_call(
        _chamfer_body,
        grid=grid,
        in_specs=[
            pl.BlockSpec((1, TILE_I, 16), lambda b, it: (b, it, 0)),
            pl.BlockSpec((1, 16, M), lambda b, it: (b, 0, 0)),
        ],
        out_specs=[
            pl.BlockSpec((1, 1, TILE_I),
                         lambda b, it: (b * (N // TILE_I) + it, 0, 0)),
            pl.BlockSpec((1, 1, M), lambda b, it: (b, 0, 0)),
        ],
        out_shape=[
            jax.ShapeDtypeStruct((B * (N // TILE_I), 1, TILE_I), jnp.float32),
            jax.ShapeDtypeStruct((B, 1, M), jnp.float32),
        ],
        interpret=interpret,
    )(lhs, rhs)
    return jnp.mean(minp) + jnp.mean(mint)


# speedup vs baseline: 1.5472x; 1.5472x over previous
import functools

import jax
import jax.numpy as jnp
from jax.experimental import pallas as pl


TILE_I = 1024


def _chamfer_body(pred_ref, tgt_ref, minp_ref, mint_ref):
    it = pl.program_id(1)
    px = pred_ref[0, :, 0:1]
    py = pred_ref[0, :, 1:2]
    pz = pred_ref[0, :, 2:3]
    tx = tgt_ref[0, 0:1, :]
    ty = tgt_ref[0, 1:2, :]
    tz = tgt_ref[0, 2:3, :]
    d = (px - tx) ** 2 + (py - ty) ** 2 + (pz - tz) ** 2
    minp_ref[0, 0, :] = jnp.min(d, axis=1)
    colmin = jnp.min(d, axis=0, keepdims=True)

    @pl.when(it == 0)
    def _init():
        mint_ref[0] = colmin

    @pl.when(it != 0)
    def _acc():
        mint_ref[0] = jnp.minimum(mint_ref[0], colmin)


@functools.partial(jax.jit, static_argnames=("interpret",))
def kernel(pred, target, interpret=False):
    B, N, _ = pred.shape
    M = target.shape[1]
    tgt_t = jnp.swapaxes(target, 1, 2)
    grid = (B, N // TILE_I)
    minp, mint = pl.pallas_call(
        _chamfer_body,
        grid=grid,
        in_specs=[
            pl.BlockSpec((1, TILE_I, 3), lambda b, it: (b, it, 0)),
            pl.BlockSpec((1, 3, M), lambda b, it: (b, 0, 0)),
        ],
        out_specs=[
            pl.BlockSpec((1, 1, TILE_I),
                         lambda b, it: (b * (N // TILE_I) + it, 0, 0)),
            pl.BlockSpec((1, 1, M), lambda b, it: (b, 0, 0)),
        ],
        out_shape=[
            jax.ShapeDtypeStruct((B * (N // TILE_I), 1, TILE_I), jnp.float32),
            jax.ShapeDtypeStruct((B, 1, M), jnp.float32),
        ],
        interpret=interpret,
    )(pred, tgt_t)
    return jnp.mean(minp) + jnp.mean(mint)


# speedup vs baseline: 1.5798x; 1.0210x over previous
import functools

import jax
import jax.numpy as jnp
from jax.experimental import pallas as pl


TILE_I = 2048


def _chamfer_body(pred_ref, tgt_ref, minp_ref, mint_ref):
    it = pl.program_id(1)
    px = pred_ref[0, :, 0:1]
    py = pred_ref[0, :, 1:2]
    pz = pred_ref[0, :, 2:3]
    tx = tgt_ref[0, 0:1, :]
    ty = tgt_ref[0, 1:2, :]
    tz = tgt_ref[0, 2:3, :]
    d = (px - tx) ** 2 + (py - ty) ** 2 + (pz - tz) ** 2
    minp_ref[0, 0, :] = jnp.min(d, axis=1)
    colmin = jnp.min(d, axis=0, keepdims=True)

    @pl.when(it == 0)
    def _init():
        mint_ref[0] = colmin

    @pl.when(it != 0)
    def _acc():
        mint_ref[0] = jnp.minimum(mint_ref[0], colmin)


@functools.partial(jax.jit, static_argnames=("interpret",))
def kernel(pred, target, interpret=False):
    B, N, _ = pred.shape
    M = target.shape[1]
    tgt_t = jnp.swapaxes(target, 1, 2)
    grid = (B, N // TILE_I)
    minp, mint = pl.pallas_call(
        _chamfer_body,
        grid=grid,
        in_specs=[
            pl.BlockSpec((1, TILE_I, 3), lambda b, it: (b, it, 0)),
            pl.BlockSpec((1, 3, M), lambda b, it: (b, 0, 0)),
        ],
        out_specs=[
            pl.BlockSpec((1, 1, TILE_I),
                         lambda b, it: (b * (N // TILE_I) + it, 0, 0)),
            pl.BlockSpec((1, 1, M), lambda b, it: (b, 0, 0)),
        ],
        out_shape=[
            jax.ShapeDtypeStruct((B * (N // TILE_I), 1, TILE_I), jnp.float32),
            jax.ShapeDtypeStruct((B, 1, M), jnp.float32),
        ],
        interpret=interpret,
    )(pred, tgt_t)
    return jnp.mean(minp) + jnp.mean(mint)


# speedup vs baseline: 1.6067x; 1.0170x over previous
import functools

import jax
import jax.numpy as jnp
from jax.experimental import pallas as pl


TILE_I = 2048


def _chamfer_body(pred_ref, tgt_ref, minp_ref, mint_ref):
    it = pl.program_id(1)
    px = pred_ref[0, :, 0:1]
    py = pred_ref[0, :, 1:2]
    pz = pred_ref[0, :, 2:3]
    tx = tgt_ref[0, 0:1, :]
    ty = tgt_ref[0, 1:2, :]
    tz = tgt_ref[0, 2:3, :]
    mtx, mty, mtz = -2.0 * tx, -2.0 * ty, -2.0 * tz
    tn = tx * tx + ty * ty + tz * tz
    pn = px * px + py * py + pz * pz
    f = (tn + px * mtx) + (py * mty + pz * mtz)
    minp_ref[0, 0, :] = jnp.min(f, axis=1) + pn[:, 0]
    colmin = jnp.min(f + pn, axis=0, keepdims=True)

    @pl.when(it == 0)
    def _init():
        mint_ref[0] = colmin

    @pl.when(it != 0)
    def _acc():
        mint_ref[0] = jnp.minimum(mint_ref[0], colmin)


@functools.partial(jax.jit, static_argnames=("interpret",))
def kernel(pred, target, interpret=False):
    B, N, _ = pred.shape
    M = target.shape[1]
    tgt_t = jnp.swapaxes(target, 1, 2)
    grid = (B, N // TILE_I)
    minp, mint = pl.pallas_call(
        _chamfer_body,
        grid=grid,
        in_specs=[
            pl.BlockSpec((1, TILE_I, 3), lambda b, it: (b, it, 0)),
            pl.BlockSpec((1, 3, M), lambda b, it: (b, 0, 0)),
        ],
        out_specs=[
            pl.BlockSpec((1, 1, TILE_I),
                         lambda b, it: (b * (N // TILE_I) + it, 0, 0)),
            pl.BlockSpec((1, 1, M), lambda b, it: (b, 0, 0)),
        ],
        out_shape=[
            jax.ShapeDtypeStruct((B * (N // TILE_I), 1, TILE_I), jnp.float32),
            jax.ShapeDtypeStruct((B, 1, M), jnp.float32),
        ],
        interpret=interpret,
    )(pred, tgt_t)
    return jnp.mean(minp) + jnp.mean(mint)
